# Initial kernel scaffold; baseline (speedup 1.0000x reference)
import functools

import jax
import jax.numpy as jnp
from jax import lax
from jax.experimental import pallas as pl
from jax.experimental.pallas import tpu as pltpu
from jax.experimental.pallas import tpu_sc as plsc

_EDGE_DIM = 16
_NUM_CORES = 2
_NUM_SUBCORES = 16
_NUM_WORKERS = _NUM_CORES * _NUM_SUBCORES
_CHUNK = 2000


@functools.partial(jax.jit, static_argnames=("num_edges",))
def _lookup(idx, table, num_edges):
    b_per_w = num_edges // _NUM_WORKERS
    n_chunks = b_per_w // _CHUNK
    mesh = plsc.VectorSubcoreMesh(core_axis_name="c", subcore_axis_name="s")

    @functools.partial(
        pl.kernel,
        mesh=mesh,
        out_type=jax.ShapeDtypeStruct((num_edges, _EDGE_DIM), jnp.float32),
        scratch_types=[
            pltpu.VMEM((_CHUNK,), jnp.int32),
            pltpu.VMEM((_CHUNK, _EDGE_DIM), jnp.float32),
            pltpu.SemaphoreType.DMA,
        ],
    )
    def k(idx_hbm, table_hbm, out_hbm, idx_v, rows_v, sem):
        wid = lax.axis_index("s") * _NUM_CORES + lax.axis_index("c")
        base = wid * b_per_w

        def body(i, carry):
            off = base + i * _CHUNK
            pltpu.sync_copy(idx_hbm.at[pl.ds(off, _CHUNK)], idx_v)
            pltpu.async_copy(table_hbm.at[idx_v], rows_v, sem).wait()
            pltpu.sync_copy(rows_v, out_hbm.at[pl.ds(off, _CHUNK)])
            return carry

        lax.fori_loop(0, n_chunks, body, 0)

    return k(idx, table)


def kernel(edge_types, table):
    batch, num_edges = edge_types.shape
    idx = edge_types.reshape(num_edges).astype(jnp.int32)
    out = _lookup(idx, table, num_edges)
    return out.reshape(batch, num_edges, _EDGE_DIM)


# speedup vs baseline: 7.1755x; 7.1755x over previous
import functools

import jax
import jax.numpy as jnp
from jax import lax
from jax.experimental import pallas as pl
from jax.experimental.pallas import tpu as pltpu
from jax.experimental.pallas import tpu_sc as plsc

_EDGE_DIM = 16
_NUM_CORES = 2
_NUM_SUBCORES = 16
_NUM_WORKERS = _NUM_CORES * _NUM_SUBCORES
_CHUNK = 2000


@functools.partial(jax.jit, static_argnames=("num_edges",))
def _lookup(idx, table, num_edges):
    b_per_w = num_edges // _NUM_WORKERS
    n_chunks = b_per_w // _CHUNK
    groups = _CHUNK // 16
    mesh = plsc.VectorSubcoreMesh(core_axis_name="c", subcore_axis_name="s")

    @functools.partial(
        pl.kernel,
        mesh=mesh,
        out_type=jax.ShapeDtypeStruct((num_edges, _EDGE_DIM), jnp.float32),
        scratch_types=[
            pltpu.VMEM((6, _EDGE_DIM), jnp.float32),
            pltpu.VMEM((_CHUNK,), jnp.int32),
            pltpu.VMEM((_CHUNK, _EDGE_DIM), jnp.float32),
            pltpu.SemaphoreType.DMA,
        ],
        compiler_params=pltpu.CompilerParams(
            use_tc_tiling_on_sc=False, needs_layout_passes=False
        ),
    )
    def k(idx_hbm, table_hbm, out_hbm, table_v, idx_v, rows_v, sem):
        wid = lax.axis_index("s") * _NUM_CORES + lax.axis_index("c")
        base = wid * b_per_w

        pltpu.sync_copy(table_hbm, table_v)
        lanes = lax.iota(jnp.int32, 16)

        def body(i, carry):
            off = base + i * _CHUNK
            pltpu.sync_copy(idx_hbm.at[pl.ds(off, _CHUNK)], idx_v)

            def expand(g):
                rows = idx_v[pl.ds(g * 16, 16)]
                e_idx = lanes + g * 16
                for j in range(_EDGE_DIM):
                    col = jnp.full((16,), j, jnp.int32)
                    val = plsc.load_gather(table_v, [rows, col])
                    plsc.store_scatter(rows_v, [e_idx, col], val)

            plsc.parallel_loop(0, groups, 1, unroll=2, carry=None)(expand)
            pltpu.sync_copy(rows_v, out_hbm.at[pl.ds(off, _CHUNK)])
            return carry

        lax.fori_loop(0, n_chunks, body, 0)

    return k(idx, table)


def kernel(edge_types, table):
    batch, num_edges = edge_types.shape
    idx = edge_types.reshape(num_edges).astype(jnp.int32)
    out = _lookup(idx, table, num_edges)
    return out.reshape(batch, num_edges, _EDGE_DIM)


# speedup vs baseline: 8.7905x; 1.2251x over previous
import functools

import jax
import jax.numpy as jnp
from jax import lax
from jax.experimental import pallas as pl
from jax.experimental.pallas import tpu as pltpu
from jax.experimental.pallas import tpu_sc as plsc

_EDGE_DIM = 16
_NUM_CORES = 2
_NUM_SUBCORES = 16
_NUM_WORKERS = _NUM_CORES * _NUM_SUBCORES
_CHUNK = 2000
_NSTORE = 4


@functools.partial(jax.jit, static_argnames=("num_edges",))
def _lookup(idx, table, num_edges):
    b_per_w = num_edges // _NUM_WORKERS
    n_chunks = b_per_w // _CHUNK
    sub = _CHUNK // _NSTORE
    mesh = plsc.VectorSubcoreMesh(core_axis_name="c", subcore_axis_name="s")

    @functools.partial(
        pl.kernel,
        mesh=mesh,
        out_type=jax.ShapeDtypeStruct((num_edges, _EDGE_DIM), jnp.float32),
        scratch_types=[
            pltpu.VMEM((6, _EDGE_DIM), jnp.float32),
            pltpu.VMEM_SHARED((16 * 6, _EDGE_DIM), jnp.float32),
            pltpu.VMEM((_CHUNK,), jnp.int32),
            pltpu.VMEM((_CHUNK, _EDGE_DIM), jnp.float32),
            pltpu.SemaphoreType.DMA,
            pltpu.SemaphoreType.DMA,
            pltpu.SemaphoreType.DMA,
            pltpu.SemaphoreType.DMA,
            pltpu.SemaphoreType.DMA,
        ],
        compiler_params=pltpu.CompilerParams(use_tc_tiling_on_sc=False),
    )
    def k(idx_hbm, table_hbm, out_hbm, table_v, table_sp, idx_v, rows_v,
          gsem, ssem0, ssem1, ssem2, ssem3):
        ssems = (ssem0, ssem1, ssem2, ssem3)
        sid = lax.axis_index("s")
        wid = sid * _NUM_CORES + lax.axis_index("c")
        base = wid * b_per_w

        @pl.when(sid == 0)
        def _():
            pltpu.sync_copy(table_hbm, table_v)
            for r in range(16):
                pltpu.sync_copy(table_v, table_sp.at[pl.ds(r * 6, 6)])

        plsc.subcore_barrier()
        my_table = table_sp.at[pl.ds(sid * 6, 6)]

        def body(i, carry):
            off = base + i * _CHUNK
            pltpu.sync_copy(idx_hbm.at[pl.ds(off, _CHUNK)], idx_v)
            pltpu.async_copy(my_table.at[idx_v], rows_v, gsem).wait()
            handles = [
                pltpu.async_copy(
                    rows_v.at[pl.ds(s * sub, sub)],
                    out_hbm.at[pl.ds(off + s * sub, sub)],
                    ssems[s],
                )
                for s in range(_NSTORE)
            ]
            for h in handles:
                h.wait()
            return carry

        lax.fori_loop(0, n_chunks, body, 0)

    return k(idx, table)


def kernel(edge_types, table):
    batch, num_edges = edge_types.shape
    idx = edge_types.reshape(num_edges).astype(jnp.int32)
    out = _lookup(idx, table, num_edges)
    return out.reshape(batch, num_edges, _EDGE_DIM)


# speedup vs baseline: 8.8022x; 1.0013x over previous
import functools

import jax
import jax.numpy as jnp
from jax import lax
from jax.experimental import pallas as pl
from jax.experimental.pallas import tpu as pltpu
from jax.experimental.pallas import tpu_sc as plsc

_EDGE_DIM = 16
_NUM_CORES = 2
_NUM_SUBCORES = 16
_NUM_WORKERS = _NUM_CORES * _NUM_SUBCORES
_CHUNK = 2000


@jax.jit
def _lookup(idx, table):
    batch, num_edges = idx.shape
    b_per_w = num_edges // _NUM_WORKERS
    n_chunks = b_per_w // _CHUNK
    mesh = plsc.VectorSubcoreMesh(core_axis_name="c", subcore_axis_name="s")

    @functools.partial(
        pl.kernel,
        mesh=mesh,
        out_type=jax.ShapeDtypeStruct((batch, num_edges, _EDGE_DIM), jnp.float32),
        scratch_types=[
            pltpu.VMEM((6, _EDGE_DIM), jnp.float32),
            pltpu.VMEM_SHARED((16 * 6, _EDGE_DIM), jnp.float32),
            pltpu.VMEM((_CHUNK,), jnp.int32),
            pltpu.VMEM((_CHUNK, _EDGE_DIM), jnp.float32),
            pltpu.SemaphoreType.DMA,
        ],
        compiler_params=pltpu.CompilerParams(use_tc_tiling_on_sc=False),
    )
    def k(idx_hbm, table_hbm, out_hbm, table_v, table_sp, idx_v, rows_v, sem):
        sid = lax.axis_index("s")
        wid = sid * _NUM_CORES + lax.axis_index("c")
        base = wid * b_per_w

        @pl.when(sid == 0)
        def _():
            pltpu.sync_copy(table_hbm, table_v)
            for r in range(16):
                pltpu.sync_copy(table_v, table_sp.at[pl.ds(r * 6, 6)])

        plsc.subcore_barrier()
        my_table = table_sp.at[pl.ds(sid * 6, 6)]

        def body(i, carry):
            off = base + i * _CHUNK
            pltpu.sync_copy(idx_hbm.at[0, pl.ds(off, _CHUNK)], idx_v)
            pltpu.async_copy(my_table.at[idx_v], rows_v, sem).wait()
            pltpu.sync_copy(rows_v, out_hbm.at[0, pl.ds(off, _CHUNK)])
            return carry

        lax.fori_loop(0, n_chunks, body, 0)

    return k(idx, table)


def kernel(edge_types, table):
    if edge_types.dtype != jnp.int32:
        edge_types = edge_types.astype(jnp.int32)
    return _lookup(edge_types, table)
